# baseline (speedup 1.0000x reference)
import functools

import jax
import jax.numpy as jnp
from jax import lax
from jax.experimental import pallas as pl
from jax.experimental.pallas import tpu as pltpu
from jax.experimental.pallas import tpu_sc as plsc

_NC = 2
_NS = 16
_L = 16
_NW = _NC * _NS


@functools.lru_cache(maxsize=None)
def _deg_kernel(N, EP):
    mesh = plsc.VectorSubcoreMesh(core_axis_name="c", subcore_axis_name="s")

    @functools.partial(
        pl.kernel,
        out_type=jax.ShapeDtypeStruct((_NW, N), jnp.float32),
        mesh=mesh,
        scratch_types=[
            pltpu.VMEM((EP,), jnp.int32),
            pltpu.VMEM((N,), jnp.float32),
        ],
        compiler_params=pltpu.CompilerParams(needs_layout_passes=False),
    )
    def deg(dst_hbm, degp_hbm, dstv, degloc):
        cid = lax.axis_index("c")
        sid = lax.axis_index("s")
        wid = cid * _NS + sid

        zero = jnp.zeros((_L,), jnp.float32)

        def zb(i, carry):
            degloc[pl.ds(i * _L, _L)] = zero
            return carry

        lax.fori_loop(0, N // _L, zb, 0)

        pltpu.sync_copy(dst_hbm.at[wid], dstv)

        ones = jnp.ones((_L,), jnp.float32)

        def eb(i, carry):
            idx = dstv[pl.ds(i * _L, _L)]
            plsc.addupdate_scatter(degloc, [idx], ones)
            return carry

        lax.fori_loop(0, EP // _L, eb, 0)

        pltpu.sync_copy(degloc, degp_hbm.at[wid])

    return deg


_NBUF = 2


@functools.lru_cache(maxsize=None)
def _edge_kernel(N, D, B, NBUF, EPP):
    RP0 = (N // _NS) // 8 * 8
    RPL = N - (_NS - 1) * RP0
    NBt = EPP // B
    assert RPL % 8 == 0 and NBt * B == EPP and NBt >= 4 and B % 8 == 0
    Na = N + _NW
    mesh = plsc.VectorSubcoreMesh(core_axis_name="c", subcore_axis_name="s")

    @functools.partial(
        pl.kernel,
        out_type=jax.ShapeDtypeStruct((_NC, N, D), jnp.float32),
        mesh=mesh,
        scratch_types=[
            pltpu.VMEM((EPP,), jnp.int32),
            pltpu.VMEM((NBt, B), jnp.int32),
            pltpu.VMEM((NBUF, B, D), jnp.float32),
            pltpu.VMEM_SHARED((Na, D), jnp.float32),
            pltpu.SemaphoreType.DMA((NBUF,)),
            pltpu.SemaphoreType.DMA((NBUF,)),
        ],
        compiler_params=pltpu.CompilerParams(needs_layout_passes=False),
    )
    def edge(u_hbm, z_hbm, src_hbm, dst_hbm, aggp_hbm, srcv, dstv, rows, acc,
             gsem, ssem):
        cid = lax.axis_index("c")
        sid = lax.axis_index("s")
        wid = cid * _NS + sid

        def for_my_rows(do_copy):
            @pl.when(sid < _NS - 1)
            def _():
                do_copy(pl.multiple_of(sid * RP0, 8), RP0)

            @pl.when(sid == _NS - 1)
            def _():
                do_copy((_NS - 1) * RP0, RPL)

        def init_copy(r0, n):
            @pl.when(cid == 0)
            def _():
                pltpu.sync_copy(u_hbm.at[pl.ds(r0, n)], acc.at[pl.ds(r0, n)])

            @pl.when(cid != 0)
            def _():
                pltpu.sync_copy(z_hbm.at[pl.ds(r0, n)], acc.at[pl.ds(r0, n)])

        for_my_rows(init_copy)
        pltpu.sync_copy(src_hbm.at[wid], srcv)
        pltpu.sync_copy(dst_hbm.at[wid], dstv)
        plsc.subcore_barrier()

        def start_gather(j, b):
            pltpu.async_copy(u_hbm.at[srcv.at[pl.ds(j * B, B)]], rows.at[b],
                             gsem.at[b])

        def wait_gather(j, b):
            pltpu.make_async_copy(u_hbm.at[srcv.at[pl.ds(j * B, B)]],
                                  rows.at[b], gsem.at[b]).wait()

        def start_scatter(j, b):
            pltpu.async_copy(rows.at[b], acc.at[dstv.at[j]], ssem.at[b],
                             add=True)

        def wait_scatter(j, b):
            pltpu.make_async_copy(rows.at[b], acc.at[dstv.at[j]],
                                  ssem.at[b]).wait()

        start_gather(0, 0)
        wait_gather(0, 0)
        start_scatter(0, 0)
        start_gather(1, 1)

        def body(j, carry):
            b = lax.rem(j, 2)
            pb = 1 - b
            wait_gather(j, b)
            start_scatter(j, b)
            wait_scatter(j - 1, pb)
            start_gather(j + 1, pb)
            return carry

        lax.fori_loop(1, NBt - 1, body, 0)

        bl = (NBt - 1) % 2
        wait_gather(NBt - 1, bl)
        start_scatter(NBt - 1, bl)
        wait_scatter(NBt - 2, 1 - bl)
        wait_scatter(NBt - 1, bl)

        plsc.subcore_barrier()

        def out_copy(r0, n):
            pltpu.sync_copy(acc.at[pl.ds(r0, n)], aggp_hbm.at[cid, pl.ds(r0, n)])

        for_my_rows(out_copy)

    return edge


def _tc_first(degp, x, W0):
    N, D = x.shape

    def body(degp_ref, x_ref, W0_ref, dinv_ref, u_ref):
        dp = degp_ref[...]
        ones = jnp.ones((dp.shape[0], 1), jnp.float32)
        deg = 1.0 + lax.dot_general(dp, ones, (((0,), (0,)), ((), ())),
                                    preferred_element_type=jnp.float32)
        dinv = lax.rsqrt(deg)
        dinv_ref[...] = dinv
        t = lax.dot_general(x_ref[...], W0_ref[...], (((1,), (1,)), ((), ())),
                            preferred_element_type=jnp.float32)
        u_ref[...] = t * dinv

    return pl.pallas_call(
        body,
        out_shape=(jax.ShapeDtypeStruct((N, 1), jnp.float32),
                   jax.ShapeDtypeStruct((N, D), jnp.float32)),
    )(degp, x, W0)


def _tc_mid(aggp, dinv, b, g, be, Wn):
    _, N, D = aggp.shape

    def body(aggp_ref, dinv_ref, b_ref, g_ref, be_ref, Wn_ref, un_ref):
        dinv = dinv_ref[...]
        out = (aggp_ref[0] + aggp_ref[1]) * dinv + b_ref[...]
        m = jnp.mean(out, axis=0, keepdims=True)
        c = out - m
        v = jnp.mean(c * c, axis=0, keepdims=True)
        y = c * lax.rsqrt(v + 1e-5) * g_ref[...] + be_ref[...]
        y = jnp.where(y >= 0, y, 0.01 * y)
        t = lax.dot_general(y, Wn_ref[...], (((1,), (1,)), ((), ())),
                            preferred_element_type=jnp.float32)
        un_ref[...] = t * dinv

    return pl.pallas_call(
        body,
        out_shape=jax.ShapeDtypeStruct((N, D), jnp.float32),
    )(aggp, dinv, b.reshape(1, D), g.reshape(1, D), be.reshape(1, D), Wn)


def _tc_last(aggp, dinv, b, g, be):
    _, N, D = aggp.shape

    def body(aggp_ref, dinv_ref, b_ref, g_ref, be_ref, y_ref):
        out = (aggp_ref[0] + aggp_ref[1]) * dinv_ref[...] + b_ref[...]
        m = jnp.mean(out, axis=0, keepdims=True)
        c = out - m
        v = jnp.mean(c * c, axis=0, keepdims=True)
        y_ref[...] = c * lax.rsqrt(v + 1e-5) * g_ref[...] + be_ref[...]

    return pl.pallas_call(
        body,
        out_shape=jax.ShapeDtypeStruct((N, D), jnp.float32),
    )(aggp, dinv, b.reshape(1, D), g.reshape(1, D), be.reshape(1, D))


def kernel(x, edge_index, W0, b0, g0, be0, W1, b1, g1, be1, W2, b2, g2, be2):
    N, D = x.shape
    E = edge_index.shape[1]
    EP = E // _NW
    B = 96
    EPP = -(-EP // B) * B
    pad = EPP - EP
    assert EP * _NW == E and EP % _L == 0 and N % _NS == 0 and N % _L == 0

    src = edge_index[0].astype(jnp.int32)
    dst = edge_index[1].astype(jnp.int32)
    srcp = jnp.pad(src.reshape(_NW, EP), ((0, 0), (0, pad)))
    dump = (N + jnp.arange(_NW, dtype=jnp.int32))[:, None]
    dstp = (jnp.pad(dst.reshape(_NW, EP), ((0, 0), (0, pad)),
                    constant_values=-1))
    dstp = jnp.where(dstp < 0, dump, dstp).reshape(_NW, EPP // B, B)
    dst2 = dst.reshape(_NW, EP)
    zeros = jnp.zeros((N, D), jnp.float32)

    degp = _deg_kernel(N, EP)(dst2)
    dinv, u = _tc_first(degp, x, W0)

    edge = _edge_kernel(N, D, B, _NBUF, EPP)
    aggp = edge(u, zeros, srcp, dstp)
    u = _tc_mid(aggp, dinv, b0, g0, be0, W1)
    aggp = edge(u, zeros, srcp, dstp)
    u = _tc_mid(aggp, dinv, b1, g1, be1, W2)
    aggp = edge(u, zeros, srcp, dstp)
    return _tc_last(aggp, dinv, b2, g2, be2)


# speedup vs baseline: 2.0295x; 2.0295x over previous
import functools

import jax
import jax.numpy as jnp
from jax import lax
from jax.experimental import pallas as pl
from jax.experimental.pallas import tpu as pltpu
from jax.experimental.pallas import tpu_sc as plsc

_NC = 2
_NS = 16
_L = 16
_NW = _NC * _NS


@functools.lru_cache(maxsize=None)
def _deg_kernel(N, EP):
    mesh = plsc.VectorSubcoreMesh(core_axis_name="c", subcore_axis_name="s")

    @functools.partial(
        pl.kernel,
        out_type=jax.ShapeDtypeStruct((_NW, N), jnp.float32),
        mesh=mesh,
        scratch_types=[
            pltpu.VMEM((EP,), jnp.int32),
            pltpu.VMEM((N,), jnp.float32),
        ],
        compiler_params=pltpu.CompilerParams(needs_layout_passes=False),
    )
    def deg(dst_hbm, degp_hbm, dstv, degloc):
        cid = lax.axis_index("c")
        sid = lax.axis_index("s")
        wid = cid * _NS + sid

        zero = jnp.zeros((_L,), jnp.float32)

        def zb(i, carry):
            degloc[pl.ds(i * _L, _L)] = zero
            return carry

        lax.fori_loop(0, N // _L, zb, 0)

        pltpu.sync_copy(dst_hbm.at[wid], dstv)

        ones = jnp.ones((_L,), jnp.float32)

        def eb(i, carry):
            idx = dstv[pl.ds(i * _L, _L)]
            plsc.addupdate_scatter(degloc, [idx], ones)
            return carry

        lax.fori_loop(0, EP // _L, eb, 0)

        pltpu.sync_copy(degloc, degp_hbm.at[wid])

    return deg


_NBUF = 3


@functools.lru_cache(maxsize=None)
def _edge_kernel(N, D, B, NBUF, EPP, padded):
    RP0 = (N // _NS) // 8 * 8
    RPL = N - (_NS - 1) * RP0
    NBt = EPP // B
    assert RPL % 8 == 0 and NBt * B == EPP and NBt >= 4 and B % 8 == 0
    Na = N + _NW if padded else N
    mesh = plsc.VectorSubcoreMesh(core_axis_name="c", subcore_axis_name="s")

    @functools.partial(
        pl.kernel,
        out_type=jax.ShapeDtypeStruct((_NC, N, D), jnp.float32),
        mesh=mesh,
        scratch_types=[
            pltpu.VMEM((EPP,), jnp.int32),
            pltpu.VMEM((EPP,), jnp.int32),
            pltpu.VMEM((NBUF, B, D), jnp.float32),
            pltpu.VMEM_SHARED((Na, D), jnp.float32),
            pltpu.SemaphoreType.DMA((NBUF,)),
            pltpu.SemaphoreType.DMA((NBUF,)),
        ],
        compiler_params=pltpu.CompilerParams(needs_layout_passes=False),
    )
    def edge(u_hbm, z_hbm, src_hbm, dst_hbm, aggp_hbm, srcv, dstv, rows, acc,
             gsem, ssem):
        cid = lax.axis_index("c")
        sid = lax.axis_index("s")
        wid = cid * _NS + sid

        def for_my_rows(do_copy):
            @pl.when(sid < _NS - 1)
            def _():
                do_copy(pl.multiple_of(sid * RP0, 8), RP0)

            @pl.when(sid == _NS - 1)
            def _():
                do_copy((_NS - 1) * RP0, RPL)

        def init_copy(r0, n):
            @pl.when(cid == 0)
            def _():
                pltpu.sync_copy(u_hbm.at[pl.ds(r0, n)], acc.at[pl.ds(r0, n)])

            @pl.when(cid != 0)
            def _():
                pltpu.sync_copy(z_hbm.at[pl.ds(r0, n)], acc.at[pl.ds(r0, n)])

        for_my_rows(init_copy)
        pltpu.sync_copy(src_hbm.at[wid], srcv)
        pltpu.sync_copy(dst_hbm.at[wid], dstv)
        plsc.subcore_barrier()

        def start_gather(j, b):
            pltpu.async_copy(u_hbm.at[srcv.at[pl.ds(j * B, B)]], rows.at[b],
                             gsem.at[b])

        def wait_gather(j, b):
            pltpu.make_async_copy(u_hbm.at[srcv.at[pl.ds(j * B, B)]],
                                  rows.at[b], gsem.at[b]).wait()

        def start_scatter(j, b):
            pltpu.async_copy(rows.at[b], acc.at[dstv.at[pl.ds(j * B, B)]],
                             ssem.at[b], add=True)

        def wait_scatter(j, b):
            pltpu.make_async_copy(rows.at[b], acc.at[dstv.at[pl.ds(j * B, B)]],
                                  ssem.at[b]).wait()

        start_gather(0, 0)
        start_gather(1, 1)
        wait_gather(0, 0)
        start_scatter(0, 0)
        start_gather(2, 2)
        wait_gather(1, 1)
        start_scatter(1, 1)
        wait_scatter(0, 0)
        start_gather(3, 0)

        def body(j, carry):
            b = lax.rem(j, 3)
            pb = lax.rem(j + 2, 3)
            wait_gather(j, b)
            start_scatter(j, b)
            wait_scatter(j - 1, pb)
            start_gather(j + 2, pb)
            return carry

        lax.fori_loop(2, NBt - 2, body, 0)

        for j in (NBt - 2, NBt - 1):
            b = j % 3
            wait_gather(j, b)
            start_scatter(j, b)
            wait_scatter(j - 1, (j + 2) % 3)
        wait_scatter(NBt - 1, (NBt - 1) % 3)

        plsc.subcore_barrier()

        def out_copy(r0, n):
            pltpu.sync_copy(acc.at[pl.ds(r0, n)], aggp_hbm.at[cid, pl.ds(r0, n)])

        for_my_rows(out_copy)

    return edge


def _tc_first(degp, x, W0):
    N, D = x.shape

    def body(degp_ref, x_ref, W0_ref, dinv_ref, u_ref):
        dp = degp_ref[...]
        ones = jnp.ones((dp.shape[0], 1), jnp.float32)
        deg = 1.0 + lax.dot_general(dp, ones, (((0,), (0,)), ((), ())),
                                    preferred_element_type=jnp.float32)
        dinv = lax.rsqrt(deg)
        dinv_ref[...] = dinv
        t = lax.dot_general(x_ref[...], W0_ref[...], (((1,), (1,)), ((), ())),
                            preferred_element_type=jnp.float32)
        u_ref[...] = t * dinv

    return pl.pallas_call(
        body,
        out_shape=(jax.ShapeDtypeStruct((N, 1), jnp.float32),
                   jax.ShapeDtypeStruct((N, D), jnp.float32)),
    )(degp, x, W0)


def _tc_mid(aggp, dinv, b, g, be, Wn):
    _, N, D = aggp.shape

    def body(aggp_ref, dinv_ref, b_ref, g_ref, be_ref, Wn_ref, un_ref):
        dinv = dinv_ref[...]
        out = (aggp_ref[0] + aggp_ref[1]) * dinv + b_ref[...]
        m = jnp.mean(out, axis=0, keepdims=True)
        c = out - m
        v = jnp.mean(c * c, axis=0, keepdims=True)
        y = c * lax.rsqrt(v + 1e-5) * g_ref[...] + be_ref[...]
        y = jnp.where(y >= 0, y, 0.01 * y)
        t = lax.dot_general(y, Wn_ref[...], (((1,), (1,)), ((), ())),
                            preferred_element_type=jnp.float32)
        un_ref[...] = t * dinv

    return pl.pallas_call(
        body,
        out_shape=jax.ShapeDtypeStruct((N, D), jnp.float32),
    )(aggp, dinv, b.reshape(1, D), g.reshape(1, D), be.reshape(1, D), Wn)


def _tc_last(aggp, dinv, b, g, be):
    _, N, D = aggp.shape

    def body(aggp_ref, dinv_ref, b_ref, g_ref, be_ref, y_ref):
        out = (aggp_ref[0] + aggp_ref[1]) * dinv_ref[...] + b_ref[...]
        m = jnp.mean(out, axis=0, keepdims=True)
        c = out - m
        v = jnp.mean(c * c, axis=0, keepdims=True)
        y_ref[...] = c * lax.rsqrt(v + 1e-5) * g_ref[...] + be_ref[...]

    return pl.pallas_call(
        body,
        out_shape=jax.ShapeDtypeStruct((N, D), jnp.float32),
    )(aggp, dinv, b.reshape(1, D), g.reshape(1, D), be.reshape(1, D))


def kernel(x, edge_index, W0, b0, g0, be0, W1, b1, g1, be1, W2, b2, g2, be2):
    N, D = x.shape
    E = edge_index.shape[1]
    EP = E // _NW
    B = 80
    EPP = -(-EP // B) * B
    pad = EPP - EP
    assert EP * _NW == E and EP % _L == 0 and N % _NS == 0 and N % _L == 0

    src = edge_index[0].astype(jnp.int32)
    dst = edge_index[1].astype(jnp.int32)
    srcp = jnp.pad(src.reshape(_NW, EP), ((0, 0), (0, pad)))
    dump = (N + jnp.arange(_NW, dtype=jnp.int32))[:, None]
    dstp = (jnp.pad(dst.reshape(_NW, EP), ((0, 0), (0, pad)),
                    constant_values=-1))
    dstp = jnp.where(dstp < 0, dump, dstp)
    dst2 = dst.reshape(_NW, EP)
    zeros = jnp.zeros((N, D), jnp.float32)

    degp = _deg_kernel(N, EP)(dst2)
    dinv, u = _tc_first(degp, x, W0)

    edge = _edge_kernel(N, D, B, _NBUF, EPP, pad > 0)
    aggp = edge(u, zeros, srcp, dstp)
    u = _tc_mid(aggp, dinv, b0, g0, be0, W1)
    aggp = edge(u, zeros, srcp, dstp)
    u = _tc_mid(aggp, dinv, b1, g1, be1, W2)
    aggp = edge(u, zeros, srcp, dstp)
    return _tc_last(aggp, dinv, b2, g2, be2)
